# baseline (speedup 1.0000x reference)
import functools

import jax
import jax.numpy as jnp
from jax import lax
from jax.experimental import pallas as pl
from jax.experimental.pallas import tpu as pltpu
from jax.experimental.pallas import tpu_sc as plsc

_NC = 2
_NS = 16
_NW = _NC * _NS


def _prep_pos_kernel(msgord_ref, fact_ref, xw2_ref, inp_ref,
                     fp_ref, pos_ref, offs_ref, tlohi_ref,
                     hist_ref, fpbuf_ref, idsbuf_ref,
                     *, order_static, num_tiles, tile_rows, num_ids):
    ph = pl.program_id(0)
    t = pl.program_id(1)
    tf = tile_rows
    ppad = offs_ref.shape[0]

    @pl.when((ph == 0) & (t == 0))
    def _():
        hist_ref[...] = jnp.zeros_like(hist_ref)

    @pl.when(ph == 0)
    def _():
        msg_to_s = msgord_ref[0]
        order_s = msgord_ref[1]
        fp = jnp.ones_like(inp_ref[0])
        for i in range(order_static):
            m = jnp.where((i < order_s) & (i != msg_to_s), 1.0, 0.0)
            fp = fp * (inp_ref[i] * m + (1.0 - m))
        fp_ref[...] = fp
        fpbuf_ref[pl.ds(t * tf, tf), :] = fp

        f0 = fact_ref[:, 0]
        f0a = f0 // 128
        f0b = f0 - f0a * 128
        iota_c = lax.broadcasted_iota(jnp.int32, (tf, 128), 1)
        ohb = (iota_c == f0b[:, None]).astype(jnp.bfloat16)
        v = jnp.dot(ohb, xw2_ref[:],
                    preferred_element_type=jnp.float32)
        mm = ((iota_c == f0a[:, None]).astype(jnp.float32) * v)
        iota_r = lax.broadcasted_iota(jnp.int32, (128, 128), 1)
        ones_col = (iota_r == 0).astype(jnp.bfloat16)
        ids_i = jnp.dot(mm.astype(jnp.bfloat16), ones_col,
                        preferred_element_type=jnp.float32)[:, 0].astype(
                            jnp.int32)
        idsbuf_ref[t, :] = ids_i

        iota_p = lax.broadcasted_iota(jnp.int32, (tf, ppad), 1)
        oh = (iota_p == ids_i[:, None]).astype(jnp.float32)
        hist_ref[0, :] = hist_ref[0, :] + jnp.sum(oh, axis=0)
        pos_ref[...] = jnp.zeros_like(pos_ref)

        @pl.when(t == num_tiles - 1)
        def _():
            hist = hist_ref[0, :]
            r = lax.broadcasted_iota(jnp.int32, (ppad, ppad), 0)
            c = lax.broadcasted_iota(jnp.int32, (ppad, ppad), 1)
            ut = (r < c).astype(jnp.float32)
            offs_f = jnp.dot(hist[None, :], ut,
                             preferred_element_type=jnp.float32)[0]
            offs_ref[...] = offs_f.astype(jnp.int32)
            hist_ref[1, :] = offs_f
            hist_ref[2, :] = jnp.zeros((ppad,), jnp.float32)
            tvals = (lax.broadcasted_iota(jnp.int32, (128, ppad), 0)
                     * tf).astype(jnp.float32)
            lo = jnp.sum(offs_f[None, :] <= tvals, axis=1).astype(jnp.int32) - 1
            hiv = jnp.sum(offs_f[None, :] <= tvals + (tf - 1.0),
                          axis=1).astype(jnp.int32) - 1
            tlohi_ref[...] = jnp.concatenate([lo[None, :], hiv[None, :]],
                                             axis=0)

    @pl.when(ph == 1)
    def _():
        fp_ref[...] = fpbuf_ref[pl.ds(t * tf, tf), :]
        ids_i = idsbuf_ref[t, :]
        iota_p = lax.broadcasted_iota(jnp.int32, (tf, ppad), 1)
        oh = (iota_p == ids_i[:, None]).astype(jnp.float32)
        offs_f = hist_ref[1, :]
        run = hist_ref[2, :]
        sub = 128
        r = lax.broadcasted_iota(jnp.int32, (sub, sub), 0)
        c = lax.broadcasted_iota(jnp.int32, (sub, sub), 1)
        lt = (c < r).astype(jnp.bfloat16)
        for s in range(tf // sub):
            ohs = oh[s * sub:(s + 1) * sub]
            ms = jnp.dot(lt, ohs.astype(jnp.bfloat16),
                         preferred_element_type=jnp.float32)
            posv = jnp.sum((ms + (run + offs_f)[None, :]) * ohs, axis=1)
            pos_ref[0, s, :] = posv.astype(jnp.int32)
            run = run + jnp.sum(ohs, axis=0)
        hist_ref[2, :] = run


def _make_permute_rows(F, D, gather):
    chunk = F // _NW
    k = chunk // 128
    mesh = plsc.VectorSubcoreMesh(core_axis_name="c", subcore_axis_name="s")

    @functools.partial(
        pl.kernel, mesh=mesh,
        out_type=jax.ShapeDtypeStruct((F, D), jnp.float32),
        scratch_types=[
            pltpu.VMEM((k, 128), jnp.int32),
            pltpu.VMEM((chunk, D), jnp.float32),
            pltpu.SemaphoreType.DMA,
        ],
    )
    def permute(rows_hbm, pos_hbm, out_hbm, idx_v, rows_v, sem):
        wid = lax.axis_index("s") * _NC + lax.axis_index("c")
        base = wid * chunk
        pltpu.sync_copy(pos_hbm.at[pl.ds(wid * k, k)], idx_v)
        if gather:
            for j in range(k):
                pltpu.async_copy(rows_hbm.at[idx_v.at[j]],
                                 rows_v.at[pl.ds(j * 128, 128)], sem).wait()
            pltpu.sync_copy(rows_v, out_hbm.at[pl.ds(base, chunk)])
        else:
            pltpu.sync_copy(rows_hbm.at[pl.ds(base, chunk)], rows_v)
            for j in range(k):
                pltpu.async_copy(rows_v.at[pl.ds(j * 128, 128)],
                                 out_hbm.at[idx_v.at[j]], sem).wait()

    return permute


def _seg_mm_kernel(offs_ref, tlohi_ref, offsv_ref, fps_ref, params_ref,
                   bias_ref, out_ref, *, tile_rows, num_ids, kc):
    t = pl.program_id(0)
    tf = tile_rows
    ppad = offsv_ref.shape[0]
    lo = tlohi_ref[0, t]
    hi = tlohi_ref[1, t]
    fpsb = fps_ref[...].astype(jnp.bfloat16)
    r_glob = lax.broadcasted_iota(jnp.int32, (tf,), 0) + t * tf
    offs_v = offsv_ref[:]
    idrow = jnp.sum((offs_v[None, :] <= r_glob[:, None]).astype(jnp.int32),
                    axis=1) - 1
    iota_p = lax.broadcasted_iota(jnp.int32, (tf, ppad), 1)
    oh_s = (iota_p == idrow[:, None]).astype(jnp.bfloat16)
    acc0 = jnp.dot(oh_s, bias_ref[:], preferred_element_type=jnp.float32)

    def chunk(c, acc):
        s = lo + c * kc
        p0 = jnp.minimum(s, num_ids - kc)
        delta = s - p0
        w2 = params_ref[pl.ds(p0, kc)].reshape(kc * fpsb.shape[1],
                                               -1).astype(jnp.bfloat16)
        cols = []
        for k in range(kc):
            o0 = offs_ref[p0 + k]
            o1 = offs_ref[p0 + k + 1]
            mk = (r_glob >= o0) & (r_glob < o1) & (k >= delta)
            cols.append(fpsb * mk.astype(jnp.bfloat16)[:, None])
        xx = jnp.concatenate(cols, axis=1)
        return acc + jnp.dot(xx, w2, preferred_element_type=jnp.float32)

    nch = (hi - lo) // kc + 1
    out_ref[...] = lax.fori_loop(0, nch, chunk, acc0)


def kernel(x, fact, inp, msg_to, order, params, bias):
    num_ids, H, O = params.shape
    order_static, F, _ = inp.shape
    n_rows = x.shape[0]

    msgord = jnp.stack([jnp.asarray(msg_to, jnp.int32),
                        jnp.asarray(order, jnp.int32)])

    m_atoms = int(round(float(num_ids) ** 0.5))
    xw = (x[:, 1] * m_atoms + x[:, 2]).astype(jnp.float32)
    na = 8
    xw_pad = jnp.zeros((na * 128,), jnp.float32).at[:n_rows].set(xw)
    xw2 = jnp.zeros((128, 128), jnp.bfloat16).at[:, :na].set(
        xw_pad.reshape(na, 128).T.astype(jnp.bfloat16))

    ppad = 256
    TF = 512
    nt = F // TF

    fp, pos2d, offs, tlohi = pl.pallas_call(
        functools.partial(_prep_pos_kernel, order_static=order_static,
                          num_tiles=nt, tile_rows=TF, num_ids=num_ids),
        grid=(2, nt),
        in_specs=[
            pl.BlockSpec(memory_space=pltpu.SMEM),
            pl.BlockSpec((TF, 2), lambda ph, t: (t, 0)),
            pl.BlockSpec((128, 128), lambda ph, t: (0, 0)),
            pl.BlockSpec((order_static, TF, H),
                         lambda ph, t: (0, t * (1 - ph), 0)),
        ],
        out_specs=[
            pl.BlockSpec((TF, H), lambda ph, t: (t, 0)),
            pl.BlockSpec((1, TF // 128, 128), lambda ph, t: (t, 0, 0)),
            pl.BlockSpec((ppad,), lambda ph, t: (0,)),
            pl.BlockSpec((2, 128), lambda ph, t: (0, 0)),
        ],
        out_shape=[
            jax.ShapeDtypeStruct((F, H), jnp.float32),
            jax.ShapeDtypeStruct((nt, TF // 128, 128), jnp.int32),
            jax.ShapeDtypeStruct((ppad,), jnp.int32),
            jax.ShapeDtypeStruct((2, 128), jnp.int32),
        ],
        scratch_shapes=[
            pltpu.VMEM((8, ppad), jnp.float32),
            pltpu.VMEM((F, H), jnp.float32),
            pltpu.VMEM((nt, TF), jnp.int32),
        ],
    )(msgord, fact, xw2, inp)
    pos2d = pos2d.reshape(F // 128, 128)

    fp_sorted = _make_permute_rows(F, H, gather=False)(fp, pos2d)

    KC = 32
    bias_pad = jnp.zeros((ppad, O), jnp.bfloat16).at[:num_ids].set(
        bias.reshape(num_ids, O).astype(jnp.bfloat16))
    out_sorted = pl.pallas_call(
        functools.partial(_seg_mm_kernel, tile_rows=TF, num_ids=num_ids,
                          kc=KC),
        grid=(nt,),
        in_specs=[
            pl.BlockSpec(memory_space=pltpu.SMEM),
            pl.BlockSpec(memory_space=pltpu.SMEM),
            pl.BlockSpec((ppad,), lambda t: (0,)),
            pl.BlockSpec((TF, H), lambda t: (t, 0)),
            pl.BlockSpec((num_ids, H, O), lambda t: (0, 0, 0)),
            pl.BlockSpec((ppad, O), lambda t: (0, 0)),
        ],
        out_specs=pl.BlockSpec((TF, O), lambda t: (t, 0)),
        out_shape=jax.ShapeDtypeStruct((F, O), jnp.float32),
    )(offs, tlohi, offs, fp_sorted, params, bias_pad)

    out = _make_permute_rows(F, O, gather=True)(out_sorted, pos2d)
    return out


# speedup vs baseline: 1.0928x; 1.0928x over previous
import functools

import jax
import jax.numpy as jnp
from jax import lax
from jax.experimental import pallas as pl
from jax.experimental.pallas import tpu as pltpu
from jax.experimental.pallas import tpu_sc as plsc

_NC = 2
_NS = 16
_NW = _NC * _NS


def _prep_pos_kernel(msgord_ref, fact_ref, xw2_ref, inp_ref,
                     fp_ref, pos_ref, offs_ref, tlohi_ref,
                     hist_ref, fpbuf_ref, idsbuf_ref,
                     *, order_static, num_tiles, tile_rows, num_ids):
    ph = pl.program_id(0)
    t = pl.program_id(1)
    tf = tile_rows
    ppad = offs_ref.shape[0]

    @pl.when((ph == 0) & (t == 0))
    def _():
        hist_ref[...] = jnp.zeros_like(hist_ref)

    @pl.when(ph == 0)
    def _():
        msg_to_s = msgord_ref[0]
        order_s = msgord_ref[1]
        fp = jnp.ones_like(inp_ref[0])
        for i in range(order_static):
            m = jnp.where((i < order_s) & (i != msg_to_s), 1.0, 0.0)
            fp = fp * (inp_ref[i] * m + (1.0 - m))
        fp_ref[...] = fp
        fpbuf_ref[pl.ds(t * tf, tf), :] = fp

        f0 = fact_ref[:, 0]
        f0a = f0 // 128
        f0b = f0 - f0a * 128
        iota_c = lax.broadcasted_iota(jnp.int32, (tf, 128), 1)
        ohb = (iota_c == f0b[:, None]).astype(jnp.bfloat16)
        v = jnp.dot(ohb, xw2_ref[:],
                    preferred_element_type=jnp.float32)
        mm = ((iota_c == f0a[:, None]).astype(jnp.float32) * v)
        iota_r = lax.broadcasted_iota(jnp.int32, (128, 128), 1)
        ones_col = (iota_r == 0).astype(jnp.bfloat16)
        ids_i = jnp.dot(mm.astype(jnp.bfloat16), ones_col,
                        preferred_element_type=jnp.float32)[:, 0].astype(
                            jnp.int32)
        idsbuf_ref[t, :] = ids_i

        iota_p = lax.broadcasted_iota(jnp.int32, (tf, ppad), 1)
        oh = (iota_p == ids_i[:, None]).astype(jnp.float32)
        hist_ref[0, :] = hist_ref[0, :] + jnp.sum(oh, axis=0)
        pos_ref[...] = jnp.zeros_like(pos_ref)

        @pl.when(t == num_tiles - 1)
        def _():
            hist = hist_ref[0, :]
            r = lax.broadcasted_iota(jnp.int32, (ppad, ppad), 0)
            c = lax.broadcasted_iota(jnp.int32, (ppad, ppad), 1)
            ut = (r < c).astype(jnp.float32)
            offs_f = jnp.dot(hist[None, :], ut,
                             preferred_element_type=jnp.float32)[0]
            offs_ref[...] = offs_f.astype(jnp.int32)
            hist_ref[1, :] = offs_f
            hist_ref[2, :] = jnp.zeros((ppad,), jnp.float32)
            tvals = (lax.broadcasted_iota(jnp.int32, (128, ppad), 0)
                     * tf).astype(jnp.float32)
            lo = jnp.sum(offs_f[None, :] <= tvals, axis=1).astype(jnp.int32) - 1
            hiv = jnp.sum(offs_f[None, :] <= tvals + (tf - 1.0),
                          axis=1).astype(jnp.int32) - 1
            tlohi_ref[...] = jnp.concatenate([lo[None, :], hiv[None, :]],
                                             axis=0)

    @pl.when(ph == 1)
    def _():
        fp_ref[...] = fpbuf_ref[pl.ds(t * tf, tf), :]
        ids_i = idsbuf_ref[t, :]
        iota_p = lax.broadcasted_iota(jnp.int32, (tf, ppad), 1)
        oh = (iota_p == ids_i[:, None]).astype(jnp.float32)
        offs_f = hist_ref[1, :]
        run = hist_ref[2, :]
        sub = 128
        r = lax.broadcasted_iota(jnp.int32, (sub, sub), 0)
        c = lax.broadcasted_iota(jnp.int32, (sub, sub), 1)
        lt = (c < r).astype(jnp.bfloat16)
        for s in range(tf // sub):
            ohs = oh[s * sub:(s + 1) * sub]
            ms = jnp.dot(lt, ohs.astype(jnp.bfloat16),
                         preferred_element_type=jnp.float32)
            posv = jnp.sum((ms + (run + offs_f)[None, :]) * ohs, axis=1)
            pos_ref[0, s, :] = posv.astype(jnp.int32)
            run = run + jnp.sum(ohs, axis=0)
        hist_ref[2, :] = run


def _make_permute_rows(F, D, gather):
    chunk = F // _NW
    k = chunk // 128
    mesh = plsc.VectorSubcoreMesh(core_axis_name="c", subcore_axis_name="s")

    @functools.partial(
        pl.kernel, mesh=mesh,
        out_type=jax.ShapeDtypeStruct((F, D), jnp.float32),
        scratch_types=[
            pltpu.VMEM((k, 128), jnp.int32),
            pltpu.VMEM((chunk, D), jnp.float32),
            pltpu.SemaphoreType.DMA,
        ],
    )
    def permute(rows_hbm, pos_hbm, out_hbm, idx_v, rows_v, sem):
        wid = lax.axis_index("s") * _NC + lax.axis_index("c")
        base = wid * chunk
        pltpu.sync_copy(pos_hbm.at[pl.ds(wid * k, k)], idx_v)
        if gather:
            for j in range(k):
                pltpu.async_copy(rows_hbm.at[idx_v.at[j]],
                                 rows_v.at[pl.ds(j * 128, 128)], sem).wait()
            pltpu.sync_copy(rows_v, out_hbm.at[pl.ds(base, chunk)])
        else:
            pltpu.sync_copy(rows_hbm.at[pl.ds(base, chunk)], rows_v)
            for j in range(k):
                pltpu.async_copy(rows_v.at[pl.ds(j * 128, 128)],
                                 out_hbm.at[idx_v.at[j]], sem).wait()

    return permute


def _seg_mm_kernel(offs_ref, tlohi_ref, offsv_ref, fps_ref, params_ref,
                   bias_ref, out_ref, *, tile_rows, num_ids, kc):
    t = pl.program_id(0)
    tf = tile_rows
    ppad = offsv_ref.shape[0]
    lo = tlohi_ref[0, t]
    hi = tlohi_ref[1, t]
    fpsb = fps_ref[...].astype(jnp.bfloat16)
    r_glob = lax.broadcasted_iota(jnp.int32, (tf,), 0) + t * tf
    offs_v = offsv_ref[:]
    idrow = jnp.sum((offs_v[None, :] <= r_glob[:, None]).astype(jnp.int32),
                    axis=1) - 1
    iota_p = lax.broadcasted_iota(jnp.int32, (tf, ppad), 1)
    oh_s = (iota_p == idrow[:, None]).astype(jnp.bfloat16)
    acc0 = jnp.dot(oh_s, bias_ref[:], preferred_element_type=jnp.float32)

    def chunk(c, acc):
        s = lo + c * kc
        p0 = jnp.minimum(s, num_ids - kc)
        delta = s - p0
        w2 = params_ref[pl.ds(p0, kc)].reshape(kc * fpsb.shape[1],
                                               -1).astype(jnp.bfloat16)
        cols = []
        for k in range(kc):
            o0 = offs_ref[p0 + k]
            o1 = offs_ref[p0 + k + 1]
            mk = (r_glob >= o0) & (r_glob < o1) & (k >= delta)
            cols.append(fpsb * mk.astype(jnp.bfloat16)[:, None])
        xx = jnp.concatenate(cols, axis=1)
        return acc + jnp.dot(xx, w2, preferred_element_type=jnp.float32)

    nch = (hi - lo) // kc + 1
    out_ref[...] = lax.fori_loop(0, nch, chunk, acc0)


def kernel(x, fact, inp, msg_to, order, params, bias):
    num_ids, H, O = params.shape
    order_static, F, _ = inp.shape
    n_rows = x.shape[0]

    msgord = jnp.stack([jnp.asarray(msg_to, jnp.int32),
                        jnp.asarray(order, jnp.int32)])

    m_atoms = int(round(float(num_ids) ** 0.5))
    xw = (x[:, 1] * m_atoms + x[:, 2]).astype(jnp.float32)
    na = 8
    xw_pad = jnp.zeros((na * 128,), jnp.float32).at[:n_rows].set(xw)
    xw2 = jnp.zeros((128, 128), jnp.bfloat16).at[:, :na].set(
        xw_pad.reshape(na, 128).T.astype(jnp.bfloat16))

    ppad = 256
    TF = 512
    nt = F // TF

    fp, pos2d, offs, tlohi = pl.pallas_call(
        functools.partial(_prep_pos_kernel, order_static=order_static,
                          num_tiles=nt, tile_rows=TF, num_ids=num_ids),
        grid=(2, nt),
        in_specs=[
            pl.BlockSpec(memory_space=pltpu.SMEM),
            pl.BlockSpec((TF, 2), lambda ph, t: (t, 0)),
            pl.BlockSpec((128, 128), lambda ph, t: (0, 0)),
            pl.BlockSpec((order_static, TF, H),
                         lambda ph, t: (0, t * (1 - ph), 0)),
        ],
        out_specs=[
            pl.BlockSpec((TF, H), lambda ph, t: (t, 0)),
            pl.BlockSpec((1, TF // 128, 128), lambda ph, t: (t, 0, 0)),
            pl.BlockSpec((ppad,), lambda ph, t: (0,)),
            pl.BlockSpec((2, 128), lambda ph, t: (0, 0)),
        ],
        out_shape=[
            jax.ShapeDtypeStruct((F, H), jnp.float32),
            jax.ShapeDtypeStruct((nt, TF // 128, 128), jnp.int32),
            jax.ShapeDtypeStruct((ppad,), jnp.int32),
            jax.ShapeDtypeStruct((2, 128), jnp.int32),
        ],
        scratch_shapes=[
            pltpu.VMEM((8, ppad), jnp.float32),
            pltpu.VMEM((F, H), jnp.float32),
            pltpu.VMEM((nt, TF), jnp.int32),
        ],
    )(msgord, fact, xw2, inp)
    pos2d = pos2d.reshape(F // 128, 128)

    fp_sorted = _make_permute_rows(F, H, gather=False)(fp, pos2d)

    KC = 16
    bias_pad = jnp.zeros((ppad, O), jnp.bfloat16).at[:num_ids].set(
        bias.reshape(num_ids, O).astype(jnp.bfloat16))
    out_sorted = pl.pallas_call(
        functools.partial(_seg_mm_kernel, tile_rows=TF, num_ids=num_ids,
                          kc=KC),
        grid=(nt,),
        in_specs=[
            pl.BlockSpec(memory_space=pltpu.SMEM),
            pl.BlockSpec(memory_space=pltpu.SMEM),
            pl.BlockSpec((ppad,), lambda t: (0,)),
            pl.BlockSpec((TF, H), lambda t: (t, 0)),
            pl.BlockSpec((num_ids, H, O), lambda t: (0, 0, 0)),
            pl.BlockSpec((ppad, O), lambda t: (0, 0)),
        ],
        out_specs=pl.BlockSpec((TF, O), lambda t: (t, 0)),
        out_shape=jax.ShapeDtypeStruct((F, O), jnp.float32),
    )(offs, tlohi, offs, fp_sorted, params, bias_pad)

    out = _make_permute_rows(F, O, gather=True)(out_sorted, pos2d)
    return out
